# baseline (speedup 1.0000x reference)
import jax
import jax.numpy as jnp
from jax.experimental import pallas as pl

_K = 5


def _acc_body(gt_ref, pred_ref, out_ref):
    i = pl.program_id(0)
    pred = pred_ref[...]
    g = gt_ref[...]
    rb, n = pred.shape
    col = jax.lax.broadcasted_iota(jnp.int32, (rb, n), 1)
    v = jnp.max(jnp.where(col == g, pred, -jnp.inf), axis=1, keepdims=True)
    enc = jnp.sum(jnp.where(pred > v, 1 << 16, 0)
                  + jnp.where(pred == v, 1, 0), axis=1)
    cnt_gt = enc >> 16
    cnt_eq = enc & 0xFFFF

    @pl.when(i == 0)
    def _():
        out_ref[...] = jnp.zeros((1, 1), jnp.float32)

    ambiguous = jnp.any((cnt_gt < _K) & (cnt_gt + cnt_eq - 1 >= _K))

    @pl.when(jnp.logical_not(ambiguous))
    def _():
        part = jnp.sum((cnt_gt < _K).astype(jnp.float32)).reshape(1, 1)
        out_ref[...] += part

    @pl.when(ambiguous)
    def _():
        cnt_eq_low = jnp.sum(((pred == v) & (col < g)).astype(jnp.int32),
                             axis=1)
        part = jnp.sum(((cnt_gt + cnt_eq_low) < _K)
                       .astype(jnp.float32)).reshape(1, 1)
        out_ref[...] += part


def kernel(pred, gt):
    b, n = pred.shape
    rb = 16
    grid = (b // rb,)
    out = pl.pallas_call(
        _acc_body,
        grid=grid,
        in_specs=[
            pl.BlockSpec((rb, 1), lambda i: (i, 0)),
            pl.BlockSpec((rb, n), lambda i: (i, 0)),
        ],
        out_specs=pl.BlockSpec((1, 1), lambda i: (0, 0)),
        out_shape=jax.ShapeDtypeStruct((1, 1), jnp.float32),
    )(gt.reshape(b, 1), pred)
    return out[0, 0] / b
